# Initial kernel scaffold; baseline (speedup 1.0000x reference)
import jax
import jax.numpy as jnp
from jax.experimental import pallas as pl


def kernel(x, residue_mask, W, b, v):
    raise NotImplementedError("write your pallas kernel here")


# speedup vs baseline: 5.6029x; 5.6029x over previous
import jax
import jax.numpy as jnp
from jax.experimental import pallas as pl
from jax.experimental.pallas import tpu as pltpu

_N, _D, _A, _B = 32768, 128, 64, 16
_BLK = 2048
_NBLK = _N // _BLK
_NEG_INF = float("-inf")


def _attn_body(mask_ref, x_ref, w_ref, b_ref, v_ref, out_ref, m_s, l_s, acc_s):
    i = pl.program_id(0)

    @pl.when(i == 0)
    def _():
        m_s[...] = jnp.full((1, _B), _NEG_INF, jnp.float32)
        l_s[...] = jnp.zeros((1, _B), jnp.float32)
        acc_s[...] = jnp.zeros((_B, _D), jnp.float32)

    x = x_ref[...]
    v_row = v_ref[...]
    u_col = jax.lax.dot_general(w_ref[...], v_row, (((0,), (1,)), ((), ())),
                                preferred_element_type=jnp.float32)
    c = jnp.sum(v_row * b_ref[...])
    u8 = jnp.broadcast_to(u_col, (_D, 8))
    s8 = jax.lax.dot_general(x, u8, (((1,), (0,)), ((), ())),
                             preferred_element_type=jnp.float32)
    s = s8[:, 0:1] + c

    seg = mask_ref[...]
    seg_ids = jax.lax.broadcasted_iota(jnp.int32, (1, _B), 1)
    oh = seg == seg_ids

    m_old = m_s[...]
    m_blk = jnp.max(jnp.where(oh, s, _NEG_INF), axis=0, keepdims=True)
    m_new = jnp.maximum(m_old, m_blk)
    scale = jnp.where(m_old == _NEG_INF, 0.0, jnp.exp(m_old - m_new))

    m_row = jnp.sum(jnp.where(oh, m_new, 0.0), axis=1, keepdims=True)
    p = jnp.exp(s - m_row)
    l_blk = jnp.sum(jnp.where(oh, p, 0.0), axis=0, keepdims=True)
    l_s[...] = l_s[...] * scale + l_blk
    m_s[...] = m_new

    g = jnp.where(oh, p, 0.0)
    part = jax.lax.dot_general(g, x, (((0,), (0,)), ((), ())),
                               preferred_element_type=jnp.float32)
    rows_i = jax.lax.broadcasted_iota(jnp.int32, (_B, _B), 0)
    cols_i = jax.lax.broadcasted_iota(jnp.int32, (_B, _B), 1)
    dscale = jnp.where(rows_i == cols_i, jnp.broadcast_to(scale, (_B, _B)), 0.0)
    acc_s[...] = jax.lax.dot_general(dscale, acc_s[...], (((1,), (0,)), ((), ())),
                                     preferred_element_type=jnp.float32) + part

    @pl.when(i == _NBLK - 1)
    def _():
        l = l_s[...]
        linv = jnp.where(l > 0.0, 1.0 / l, 0.0)
        dinv = jnp.where(rows_i == cols_i, jnp.broadcast_to(linv, (_B, _B)), 0.0)
        out_ref[...] = jax.lax.dot_general(dinv, acc_s[...], (((1,), (0,)), ((), ())),
                                           preferred_element_type=jnp.float32)


@jax.jit
def kernel(x, residue_mask, W, b, v):
    mask = residue_mask.astype(jnp.int32).reshape(_N, 1)
    return pl.pallas_call(
        _attn_body,
        grid=(_NBLK,),
        in_specs=[
            pl.BlockSpec((_BLK, 1), lambda i: (i, 0)),
            pl.BlockSpec((_BLK, _D), lambda i: (i, 0)),
            pl.BlockSpec((_A, _D), lambda i: (0, 0)),
            pl.BlockSpec((1, _A), lambda i: (0, 0)),
            pl.BlockSpec((1, _A), lambda i: (0, 0)),
        ],
        out_specs=pl.BlockSpec((_B, _D), lambda i: (0, 0)),
        out_shape=jax.ShapeDtypeStruct((_B, _D), jnp.float32),
        scratch_shapes=[
            pltpu.VMEM((1, _B), jnp.float32),
            pltpu.VMEM((1, _B), jnp.float32),
            pltpu.VMEM((_B, _D), jnp.float32),
        ],
        compiler_params=pltpu.CompilerParams(
            dimension_semantics=("arbitrary",),
        ),
    )(mask, x, W, b.reshape(1, _A), v.reshape(1, _A))


# speedup vs baseline: 12.6876x; 2.2645x over previous
import jax
import jax.numpy as jnp
from jax.experimental import pallas as pl
from jax.experimental.pallas import tpu as pltpu

_N, _D, _A, _B = 32768, 128, 64, 16
_BLK = 2048
_NBLK = _N // _BLK
_NEG_INF = float("-inf")


def _attn_body(mask_ref, x_ref, w_ref, b_ref, v_ref, out_ref, m_s, l_s, acc_s):
    i = pl.program_id(0)

    @pl.when(i == 0)
    def _():
        m_s[...] = jnp.full((_B, 1), _NEG_INF, jnp.float32)
        l_s[...] = jnp.zeros((_B, 1), jnp.float32)
        acc_s[...] = jnp.zeros((_B, _D), jnp.float32)

    x = x_ref[...]
    v_row = v_ref[...]
    u_row = jax.lax.dot_general(v_row, w_ref[...], (((1,), (0,)), ((), ())),
                                preferred_element_type=jnp.float32)
    c = jnp.sum(v_row * b_ref[...])
    u8 = jnp.broadcast_to(u_row, (8, _D))
    s8 = jax.lax.dot_general(u8, x, (((1,), (1,)), ((), ())),
                             preferred_element_type=jnp.float32)
    s1 = s8[0:1, :] + c

    seg = mask_ref[0]
    oh = jax.lax.broadcasted_iota(jnp.int32, (_B, _BLK), 0) == seg

    m_old = m_s[...]
    sm = jnp.where(oh, s1, _NEG_INF)
    m_blk = jnp.max(sm, axis=1, keepdims=True)
    m_new = jnp.maximum(m_old, m_blk)
    scale = jnp.where(m_old == _NEG_INF, 0.0, jnp.exp(m_old - m_new))

    pm = jnp.where(oh, jnp.exp(s1 - m_new), 0.0)
    l_blk = jnp.sum(pm, axis=1, keepdims=True)
    l_s[...] = l_s[...] * scale + l_blk
    m_s[...] = m_new

    part = jax.lax.dot_general(pm, x, (((1,), (0,)), ((), ())),
                               preferred_element_type=jnp.float32)
    acc_s[...] = acc_s[...] * scale + part

    @pl.when(i == _NBLK - 1)
    def _():
        l = l_s[...]
        linv = jnp.where(l > 0.0, 1.0 / l, 0.0)
        out_ref[...] = acc_s[...] * linv


@jax.jit
def kernel(x, residue_mask, W, b, v):
    mask = residue_mask.astype(jnp.int32).reshape(_NBLK, 1, _BLK)
    return pl.pallas_call(
        _attn_body,
        grid=(_NBLK,),
        in_specs=[
            pl.BlockSpec((1, 1, _BLK), lambda i: (i, 0, 0)),
            pl.BlockSpec((_BLK, _D), lambda i: (i, 0)),
            pl.BlockSpec((_A, _D), lambda i: (0, 0)),
            pl.BlockSpec((1, _A), lambda i: (0, 0)),
            pl.BlockSpec((1, _A), lambda i: (0, 0)),
        ],
        out_specs=pl.BlockSpec((_B, _D), lambda i: (0, 0)),
        out_shape=jax.ShapeDtypeStruct((_B, _D), jnp.float32),
        scratch_shapes=[
            pltpu.VMEM((_B, 1), jnp.float32),
            pltpu.VMEM((_B, 1), jnp.float32),
            pltpu.VMEM((_B, _D), jnp.float32),
        ],
        compiler_params=pltpu.CompilerParams(
            dimension_semantics=("arbitrary",),
        ),
    )(mask, x, W, b.reshape(1, _A), v.reshape(1, _A))
